# baseline (speedup 1.0000x reference)
import functools

import jax
import jax.numpy as jnp
from jax import lax
from jax.experimental import pallas as pl
from jax.experimental.pallas import tpu as pltpu
from jax.experimental.pallas import tpu_sc as plsc

_B = 4
_S = 8192
_D = 1024
_NW = 32
_ROWS_PER_W = _S // _NW
_CHUNK_ROWS = 8
_CHUNK = _CHUNK_ROWS * _D
_NCHUNKS = _ROWS_PER_W // _CHUNK_ROWS
_PSZ = _S * _D
_XSZ = _B * _S * _D

_mesh = plsc.VectorSubcoreMesh(core_axis_name="c", subcore_axis_name="s")


@functools.partial(
    pl.kernel,
    out_type=jax.ShapeDtypeStruct((_XSZ,), jnp.float32),
    mesh=_mesh,
    scratch_types=[
        pltpu.VMEM((_CHUNK,), jnp.float32),
        pltpu.VMEM((_B, _CHUNK), jnp.float32),
        pltpu.SemaphoreType.DMA,
        pltpu.SemaphoreType.DMA,
    ],
)
def _pos_add(x_hbm, pos_hbm, out_hbm, pos_buf, x_buf, sem_in, sem_out):
    cid = lax.axis_index("c")
    sid = lax.axis_index("s")
    wid = sid * 2 + cid
    base = wid * (_ROWS_PER_W * _D)

    def chunk(i, carry):
        off = base + i * _CHUNK
        loads = [
            pltpu.make_async_copy(
                pos_hbm.at[pl.ds(off, _CHUNK)], pos_buf, sem_in)
        ]
        for b in range(_B):
            loads.append(pltpu.make_async_copy(
                x_hbm.at[pl.ds(b * _PSZ + off, _CHUNK)], x_buf.at[b], sem_in))
        for d in loads:
            d.start()
        for d in loads:
            d.wait()

        def step(j, c2):
            sl = pl.ds(j * 16, 16)
            pv = pos_buf[sl]
            for b in range(_B):
                x_buf[b, sl] = x_buf[b, sl] + pv
            return c2

        lax.fori_loop(0, _CHUNK // 16, step, 0)

        stores = [
            pltpu.make_async_copy(
                x_buf.at[b], out_hbm.at[pl.ds(b * _PSZ + off, _CHUNK)],
                sem_out)
            for b in range(_B)
        ]
        for d in stores:
            d.start()
        for d in stores:
            d.wait()
        return carry

    lax.fori_loop(0, _NCHUNKS, chunk, 0)


@jax.jit
def kernel(x, pos_table):
    out = _pos_add(x.reshape(-1), pos_table.reshape(-1))
    return out.reshape(x.shape)


# speedup vs baseline: 1.0943x; 1.0943x over previous
import functools

import jax
import jax.numpy as jnp
from jax import lax
from jax.experimental import pallas as pl
from jax.experimental.pallas import tpu as pltpu
from jax.experimental.pallas import tpu_sc as plsc

_B = 4
_S = 8192
_D = 1024
_NW = 32
_ROWS_PER_W = _S // _NW
_CHUNK_ROWS = 8
_CHUNK = _CHUNK_ROWS * _D
_NCHUNKS = _ROWS_PER_W // _CHUNK_ROWS
_PSZ = _S * _D
_XSZ = _B * _S * _D

_mesh = plsc.VectorSubcoreMesh(core_axis_name="c", subcore_axis_name="s")


@functools.partial(
    pl.kernel,
    out_type=jax.ShapeDtypeStruct((_XSZ,), jnp.float32),
    mesh=_mesh,
    scratch_types=[
        pltpu.VMEM((2, _CHUNK), jnp.float32),
        pltpu.VMEM((2, _B, _CHUNK), jnp.float32),
        pltpu.SemaphoreType.DMA,
        pltpu.SemaphoreType.DMA,
        pltpu.SemaphoreType.DMA,
        pltpu.SemaphoreType.DMA,
    ],
)
def _pos_add(x_hbm, pos_hbm, out_hbm, pos_buf, x_buf,
             sin0, sin1, sout0, sout1):
    cid = lax.axis_index("c")
    sid = lax.axis_index("s")
    wid = sid * 2 + cid
    base = wid * (_ROWS_PER_W * _D)
    sin = (sin0, sin1)
    sout = (sout0, sout1)

    def load_descs(c, p):
        off = base + c * _CHUNK
        descs = [pltpu.make_async_copy(
            pos_hbm.at[pl.ds(off, _CHUNK)], pos_buf.at[p], sin[p])]
        for b in range(_B):
            descs.append(pltpu.make_async_copy(
                x_hbm.at[pl.ds(b * _PSZ + off, _CHUNK)],
                x_buf.at[p, b], sin[p]))
        return descs

    def store_descs(c, p):
        off = base + c * _CHUNK
        return [pltpu.make_async_copy(
            x_buf.at[p, b], out_hbm.at[pl.ds(b * _PSZ + off, _CHUNK)],
            sout[p]) for b in range(_B)]

    def compute(p):
        @plsc.parallel_loop(0, _CHUNK // 16, unroll=8)
        def _(j):
            sl = pl.ds(j * 16, 16)
            pv = pos_buf[p, sl]
            for b in range(_B):
                x_buf[p, b, sl] = x_buf[p, b, sl] + pv

    for d in load_descs(0, 0):
        d.start()
    for c in range(_NCHUNKS):
        p = c & 1
        for d in load_descs(c, p):
            d.wait()
        compute(p)
        for d in store_descs(c, p):
            d.start()
        if c + 1 < _NCHUNKS:
            if c >= 1:
                for d in store_descs(c - 1, p ^ 1):
                    d.wait()
            for d in load_descs(c + 1, p ^ 1):
                d.start()
    for d in store_descs(_NCHUNKS - 2, (_NCHUNKS - 2) & 1):
        d.wait()
    for d in store_descs(_NCHUNKS - 1, (_NCHUNKS - 1) & 1):
        d.wait()


@jax.jit
def kernel(x, pos_table):
    out = _pos_add(x.reshape(-1), pos_table.reshape(-1))
    return out.reshape(x.shape)


# speedup vs baseline: 2.7811x; 2.5415x over previous
import functools

import jax
import jax.numpy as jnp
from jax import lax
from jax.experimental import pallas as pl
from jax.experimental.pallas import tpu as pltpu
from jax.experimental.pallas import tpu_sc as plsc

_B = 4
_S = 8192
_D = 1024
_NW = 32
_ROWS_PER_W = _S // _NW
_CHUNK_ROWS = 8
_NCHUNKS = _ROWS_PER_W // _CHUNK_ROWS
_NVREG = _CHUNK_ROWS * _D // 16

_mesh = plsc.VectorSubcoreMesh(core_axis_name="c", subcore_axis_name="s")


@functools.partial(
    pl.kernel,
    out_type=jax.ShapeDtypeStruct((_B * _S, _D), jnp.float32),
    mesh=_mesh,
    scratch_types=[
        pltpu.VMEM((2, _CHUNK_ROWS, _D), jnp.float32),
        pltpu.VMEM((2, _B, _CHUNK_ROWS, _D), jnp.float32),
        pltpu.SemaphoreType.DMA,
        pltpu.SemaphoreType.DMA,
        pltpu.SemaphoreType.DMA,
        pltpu.SemaphoreType.DMA,
    ],
)
def _pos_add(x_hbm, pos_hbm, out_hbm, pos_buf, x_buf,
             sin0, sin1, sout0, sout1):
    cid = lax.axis_index("c")
    sid = lax.axis_index("s")
    wid = sid * 2 + cid
    base = wid * _ROWS_PER_W
    sin = (sin0, sin1)
    sout = (sout0, sout1)

    def load_descs(c, p):
        row = base + c * _CHUNK_ROWS
        descs = [pltpu.make_async_copy(
            pos_hbm.at[pl.ds(row, _CHUNK_ROWS)], pos_buf.at[p], sin[p])]
        for b in range(_B):
            descs.append(pltpu.make_async_copy(
                x_hbm.at[pl.ds(b * _S + row, _CHUNK_ROWS)],
                x_buf.at[p, b], sin[p]))
        return descs

    def store_descs(c, p):
        row = base + c * _CHUNK_ROWS
        return [pltpu.make_async_copy(
            x_buf.at[p, b], out_hbm.at[pl.ds(b * _S + row, _CHUNK_ROWS)],
            sout[p]) for b in range(_B)]

    def compute(p):
        @plsc.parallel_loop(0, _NVREG, unroll=8)
        def _(j):
            r = j // (_D // 16)
            sl = pl.ds((j % (_D // 16)) * 16, 16)
            pv = pos_buf[p, r, sl]
            for b in range(_B):
                x_buf[p, b, r, sl] = x_buf[p, b, r, sl] + pv

    for d in load_descs(0, 0):
        d.start()
    for c in range(_NCHUNKS):
        p = c & 1
        for d in load_descs(c, p):
            d.wait()
        compute(p)
        for d in store_descs(c, p):
            d.start()
        if c + 1 < _NCHUNKS:
            if c >= 1:
                for d in store_descs(c - 1, p ^ 1):
                    d.wait()
            for d in load_descs(c + 1, p ^ 1):
                d.start()
    for d in store_descs(_NCHUNKS - 2, (_NCHUNKS - 2) & 1):
        d.wait()
    for d in store_descs(_NCHUNKS - 1, (_NCHUNKS - 1) & 1):
        d.wait()


@jax.jit
def kernel(x, pos_table):
    out = _pos_add(x.reshape(_B * _S, _D), pos_table)
    return out.reshape(x.shape)


# speedup vs baseline: 3.8053x; 1.3683x over previous
import functools

import jax
import jax.numpy as jnp
from jax import lax
from jax.experimental import pallas as pl
from jax.experimental.pallas import tpu as pltpu
from jax.experimental.pallas import tpu_sc as plsc

_B = 4
_S = 8192
_D = 1024
_NW = 32
_ROWS_PER_W = _S // _NW
_CHUNK_ROWS = 8
_NCHUNKS = _ROWS_PER_W // _CHUNK_ROWS
_NVREG = _CHUNK_ROWS * _D // 16
_NBUF = 3

_mesh = plsc.VectorSubcoreMesh(core_axis_name="c", subcore_axis_name="s")


@functools.partial(
    pl.kernel,
    out_type=jax.ShapeDtypeStruct((_B * _S, _D), jnp.float32),
    mesh=_mesh,
    scratch_types=[
        pltpu.VMEM((_NBUF, _CHUNK_ROWS, _D), jnp.float32),
        pltpu.VMEM((_NBUF, _B, _CHUNK_ROWS, _D), jnp.float32),
        pltpu.SemaphoreType.DMA,
        pltpu.SemaphoreType.DMA,
        pltpu.SemaphoreType.DMA,
        pltpu.SemaphoreType.DMA,
        pltpu.SemaphoreType.DMA,
        pltpu.SemaphoreType.DMA,
    ],
)
def _pos_add(x_hbm, pos_hbm, out_hbm, pos_buf, x_buf,
             sin0, sin1, sin2, sout0, sout1, sout2):
    cid = lax.axis_index("c")
    sid = lax.axis_index("s")
    wid = sid * 2 + cid
    base = wid * _ROWS_PER_W
    sin = (sin0, sin1, sin2)
    sout = (sout0, sout1, sout2)

    def load_descs(c, p):
        row = base + c * _CHUNK_ROWS
        descs = [pltpu.make_async_copy(
            pos_hbm.at[pl.ds(row, _CHUNK_ROWS)], pos_buf.at[p], sin[p])]
        for b in range(_B):
            descs.append(pltpu.make_async_copy(
                x_hbm.at[pl.ds(b * _S + row, _CHUNK_ROWS)],
                x_buf.at[p, b], sin[p]))
        return descs

    def store_descs(c, p):
        row = base + c * _CHUNK_ROWS
        return [pltpu.make_async_copy(
            x_buf.at[p, b], out_hbm.at[pl.ds(b * _S + row, _CHUNK_ROWS)],
            sout[p]) for b in range(_B)]

    def compute(p):
        @plsc.parallel_loop(0, _NVREG, unroll=8)
        def _(j):
            r = j // (_D // 16)
            sl = pl.ds((j % (_D // 16)) * 16, 16)
            pv = pos_buf[p, r, sl]
            for b in range(_B):
                x_buf[p, b, r, sl] = x_buf[p, b, r, sl] + pv

    for d in load_descs(0, 0):
        d.start()
    for d in load_descs(1, 1):
        d.start()
    for c in range(_NCHUNKS):
        p = c % _NBUF
        for d in load_descs(c, p):
            d.wait()
        compute(p)
        for d in store_descs(c, p):
            d.start()
        if c + 2 < _NCHUNKS:
            q = (c + 2) % _NBUF
            if c >= 1:
                for d in store_descs(c - 1, q):
                    d.wait()
            for d in load_descs(c + 2, q):
                d.start()
    for c in range(_NCHUNKS - 3, _NCHUNKS):
        for d in store_descs(c, c % _NBUF):
            d.wait()


@jax.jit
def kernel(x, pos_table):
    out = _pos_add(x.reshape(_B * _S, _D), pos_table)
    return out.reshape(x.shape)
